# Initial kernel scaffold; baseline (speedup 1.0000x reference)
import jax
import jax.numpy as jnp
from jax.experimental import pallas as pl


def kernel(x, edge_index, W, b):
    raise NotImplementedError("write your pallas kernel here")


# speedup vs baseline: 11.8345x; 11.8345x over previous
import functools

import jax
import jax.numpy as jnp
from jax import lax
from jax.experimental import pallas as pl
from jax.experimental.pallas import tpu as pltpu
from jax.experimental.pallas import tpu_sc as plsc

N = 10000
D = 256
E = 160000
HALF = D // 2
NC = 2
NS = 16
NPAD = 10240
RPT = NPAD // NS

CH = 128
EPT = E // NS
EPTP = 10240
NCH = EPTP // CH

EW = E // (NC * NS)
EWP = 5120
NCHD = EWP // CH

BN = 512
GN = NPAD // BN

_mesh = plsc.VectorSubcoreMesh(core_axis_name="c", subcore_axis_name="s")


@functools.partial(
    pl.kernel,
    out_type=jax.ShapeDtypeStruct((NC * NPAD,), jnp.float32),
    mesh=_mesh,
    scratch_types=[
        pltpu.VMEM((NCHD, CH), jnp.int32),
        pltpu.VMEM((CH,), jnp.float32),
        pltpu.VMEM_SHARED((NPAD,), jnp.float32),
        [pltpu.SemaphoreType.DMA for _ in range(4)],
    ],
)
def _deg_kernel(dstdeg_hbm, zeros_hbm, ones_hbm, deg_hbm,
                idx_v, ones_v, deg_sh, sems):
    c = lax.axis_index("c")
    s = lax.axis_index("s")
    w = c * NS + s
    pltpu.sync_copy(zeros_hbm.at[pl.ds(s * RPT, RPT)], deg_sh.at[pl.ds(s * RPT, RPT)])
    pltpu.sync_copy(ones_hbm, ones_v)
    pltpu.sync_copy(dstdeg_hbm.at[pl.ds(w * NCHD, NCHD)], idx_v)
    plsc.subcore_barrier()

    for b in range(4):
        pltpu.async_copy(ones_v, deg_sh.at[idx_v.at[b]], sems[b], add=True)

    def body(g, carry):
        for b in range(4):
            k = g * 4 + b
            pltpu.make_async_copy(ones_v, deg_sh.at[idx_v.at[k]], sems[b]).wait()
            kn = k + 4

            @pl.when(kn < NCHD)
            def _():
                pltpu.async_copy(ones_v, deg_sh.at[idx_v.at[kn]], sems[b], add=True)

        return carry

    lax.fori_loop(0, NCHD // 4, body, 0)
    plsc.subcore_barrier()
    pltpu.sync_copy(deg_sh.at[pl.ds(s * RPT, RPT)],
                    deg_hbm.at[pl.ds(c * NPAD + s * RPT, RPT)])


def _matmul_body(x_ref, w_ref, b_ref, d0_ref, d1_ref, out_ref):
    h = jnp.dot(x_ref[...], w_ref[...], preferred_element_type=jnp.float32)
    h = h + b_ref[...]
    dinv = lax.rsqrt(d0_ref[...] + d1_ref[...] + 1.0)
    out_ref[...] = h * dinv


def _matmul_scale(x_p, w, b_row, deg0, deg1):
    return pl.pallas_call(
        _matmul_body,
        grid=(GN, NC),
        in_specs=[
            pl.BlockSpec((BN, D), lambda i, j: (i, 0)),
            pl.BlockSpec((D, HALF), lambda i, j: (0, j)),
            pl.BlockSpec((1, HALF), lambda i, j: (0, j)),
            pl.BlockSpec((BN, 1), lambda i, j: (i, 0)),
            pl.BlockSpec((BN, 1), lambda i, j: (i, 0)),
        ],
        out_specs=pl.BlockSpec((BN, HALF), lambda i, j: (j * GN + i, 0)),
        out_shape=jax.ShapeDtypeStruct((NC * NPAD, HALF), jnp.float32),
    )(x_p, w, b_row, deg0, deg1)


@functools.partial(
    pl.kernel,
    out_type=jax.ShapeDtypeStruct((NC * NPAD, HALF), jnp.float32),
    mesh=_mesh,
    scratch_types=[
        [pltpu.VMEM((CH,), jnp.int32) for _ in range(4)],
        pltpu.VMEM((NCH, CH), jnp.int32),
        [pltpu.VMEM((CH, HALF), jnp.float32) for _ in range(2)],
        pltpu.VMEM_SHARED((NPAD, HALF), jnp.float32),
        [pltpu.SemaphoreType.DMA for _ in range(4)],
        [pltpu.SemaphoreType.DMA for _ in range(2)],
    ],
)
def _scatter_kernel(hs_hbm, src2_hbm, dst2d_hbm, acc_hbm,
                    src_v, dst_v, rows_v, acc_sh, isems, gsems):
    c = lax.axis_index("c")
    s = lax.axis_index("s")
    rowbase = c * NPAD
    ebase = (c * NS + s) * EPTP
    pltpu.sync_copy(hs_hbm.at[pl.ds(rowbase + s * RPT, RPT)],
                    acc_sh.at[pl.ds(s * RPT, RPT)])
    pltpu.sync_copy(dst2d_hbm.at[pl.ds(s * NCH, NCH)], dst_v)
    plsc.subcore_barrier()

    def issue_idx(k, bi):
        pltpu.async_copy(src2_hbm.at[pl.ds(ebase + k * CH, CH)],
                         src_v[bi], isems[bi])

    def wait_idx(k, bi):
        pltpu.make_async_copy(src2_hbm.at[pl.ds(ebase + k * CH, CH)],
                              src_v[bi], isems[bi]).wait()

    def issue_gather(bi, bg):
        pltpu.async_copy(hs_hbm.at[src_v[bi]], rows_v[bg], gsems[bg])

    def wait_gather(bi, bg):
        pltpu.make_async_copy(hs_hbm.at[src_v[bi]], rows_v[bg],
                              gsems[bg]).wait()

    for k in range(4):
        issue_idx(k, k)
    for k in range(2):
        wait_idx(k, k)
        issue_gather(k, k)

    def body(g, carry):
        for b in range(4):
            k = g * 4 + b
            wait_gather(b, b % 2)
            pltpu.sync_copy(rows_v[b % 2], acc_sh.at[dst_v.at[k]], add=True)
            kn = k + 4

            @pl.when(kn < NCH)
            def _():
                issue_idx(kn, b)

            kg = k + 2

            @pl.when(kg < NCH)
            def _():
                wait_idx(kg, (b + 2) % 4)
                issue_gather((b + 2) % 4, b % 2)

        return carry

    lax.fori_loop(0, NCH // 4, body, 0)
    plsc.subcore_barrier()
    pltpu.sync_copy(acc_sh.at[pl.ds(s * RPT, RPT)],
                    acc_hbm.at[pl.ds(rowbase + s * RPT, RPT)])


def _finish_body(acc_ref, d0_ref, d1_ref, out_ref):
    dinv = lax.rsqrt(d0_ref[...] + d1_ref[...] + 1.0)
    out_ref[...] = jnp.maximum(acc_ref[...] * dinv, 0.0)


def _finish(acc, deg0, deg1):
    return pl.pallas_call(
        _finish_body,
        grid=(GN, NC),
        in_specs=[
            pl.BlockSpec((BN, HALF), lambda i, j: (j * GN + i, 0)),
            pl.BlockSpec((BN, 1), lambda i, j: (i, 0)),
            pl.BlockSpec((BN, 1), lambda i, j: (i, 0)),
        ],
        out_specs=pl.BlockSpec((BN, HALF), lambda i, j: (i, j)),
        out_shape=jax.ShapeDtypeStruct((N, D), jnp.float32),
    )(acc, deg0, deg1)


def kernel(x, edge_index, W, b):
    src = edge_index[0]
    dst = edge_index[1]
    spread = (N + jnp.arange(256, dtype=jnp.int32) % (NPAD - N))
    pad3 = jnp.broadcast_to(spread[: EPTP - EPT], (NS, EPTP - EPT))
    src_p = jnp.concatenate(
        [src.reshape(NS, EPT), jnp.zeros((NS, EPTP - EPT), jnp.int32)], axis=1
    ).reshape(-1)
    dst_p = jnp.concatenate([dst.reshape(NS, EPT), pad3], axis=1)
    src2 = jnp.concatenate([src_p, src_p + NPAD])
    dst2d = dst_p.reshape(NS * NCH, CH)
    pad1 = jnp.broadcast_to(spread[: EWP - EW], (NC * NS, EWP - EW))
    dstdeg = jnp.concatenate(
        [dst.reshape(NC * NS, EW), pad1], axis=1
    ).reshape(NC * NS * NCHD, CH)
    zeros_col = jnp.zeros((NPAD,), jnp.float32)
    ones_chunk = jnp.ones((CH,), jnp.float32)
    b_row = b.reshape(1, D)

    deg2 = _deg_kernel(dstdeg, zeros_col, ones_chunk)
    deg0 = deg2[:NPAD].reshape(NPAD, 1)
    deg1 = deg2[NPAD:].reshape(NPAD, 1)
    hs = _matmul_scale(x, W, b_row, deg0, deg1)
    acc = _scatter_kernel(hs, src2, dst2d)
    return _finish(acc, deg0, deg1)


# speedup vs baseline: 18.9628x; 1.6023x over previous
import functools

import jax
import jax.numpy as jnp
from jax import lax
from jax.experimental import pallas as pl
from jax.experimental.pallas import tpu as pltpu
from jax.experimental.pallas import tpu_sc as plsc

N = 10000
D = 256
E = 160000
HALF = D // 2
NC = 2
NS = 16
NPAD = 10240
RPT = NPAD // NS

CH = 80
EPT = E // NS
EPTP = EPT
NCH = EPTP // CH
CHD = 128

EW = E // (NC * NS)
EWP = 5120
NCHD = EWP // CHD

BN = 512
GN = NPAD // BN

_mesh = plsc.VectorSubcoreMesh(core_axis_name="c", subcore_axis_name="s")


@functools.partial(
    pl.kernel,
    out_type=jax.ShapeDtypeStruct((NC * NPAD,), jnp.float32),
    mesh=_mesh,
    scratch_types=[
        pltpu.VMEM((NCHD, CHD), jnp.int32),
        pltpu.VMEM((CHD,), jnp.float32),
        pltpu.VMEM_SHARED((NPAD,), jnp.float32),
        [pltpu.SemaphoreType.DMA for _ in range(4)],
    ],
)
def _deg_kernel(dstdeg_hbm, zeros_hbm, ones_hbm, deg_hbm,
                idx_v, ones_v, deg_sh, sems):
    c = lax.axis_index("c")
    s = lax.axis_index("s")
    w = c * NS + s
    pltpu.sync_copy(zeros_hbm.at[pl.ds(s * RPT, RPT)], deg_sh.at[pl.ds(s * RPT, RPT)])
    pltpu.sync_copy(ones_hbm, ones_v)
    pltpu.sync_copy(dstdeg_hbm.at[pl.ds(w * NCHD, NCHD)], idx_v)
    plsc.subcore_barrier()

    for b in range(4):
        pltpu.async_copy(ones_v, deg_sh.at[idx_v.at[b]], sems[b], add=True)

    def body(g, carry):
        for b in range(4):
            k = g * 4 + b
            pltpu.make_async_copy(ones_v, deg_sh.at[idx_v.at[k]], sems[b]).wait()
            kn = k + 4

            @pl.when(kn < NCHD)
            def _():
                pltpu.async_copy(ones_v, deg_sh.at[idx_v.at[kn]], sems[b], add=True)

        return carry

    lax.fori_loop(0, NCHD // 4, body, 0)
    plsc.subcore_barrier()
    pltpu.sync_copy(deg_sh.at[pl.ds(s * RPT, RPT)],
                    deg_hbm.at[pl.ds(c * NPAD + s * RPT, RPT)])


def _matmul_body(x_ref, w_ref, b_ref, d0_ref, d1_ref, out_ref):
    h = jnp.dot(x_ref[...], w_ref[...], preferred_element_type=jnp.float32)
    h = h + b_ref[...]
    dinv = lax.rsqrt(d0_ref[...] + d1_ref[...] + 1.0)
    out_ref[...] = h * dinv


def _matmul_scale(x_p, w, b_row, deg0, deg1):
    return pl.pallas_call(
        _matmul_body,
        grid=(GN, NC),
        in_specs=[
            pl.BlockSpec((BN, D), lambda i, j: (i, 0)),
            pl.BlockSpec((D, HALF), lambda i, j: (0, j)),
            pl.BlockSpec((1, HALF), lambda i, j: (0, j)),
            pl.BlockSpec((BN, 1), lambda i, j: (i, 0)),
            pl.BlockSpec((BN, 1), lambda i, j: (i, 0)),
        ],
        out_specs=pl.BlockSpec((BN, HALF), lambda i, j: (j * GN + i, 0)),
        out_shape=jax.ShapeDtypeStruct((NC * NPAD, HALF), jnp.float32),
    )(x_p, w, b_row, deg0, deg1)


@functools.partial(
    pl.kernel,
    out_type=jax.ShapeDtypeStruct((NC * NPAD, HALF), jnp.float32),
    mesh=_mesh,
    scratch_types=[
        [pltpu.VMEM((CH,), jnp.int32) for _ in range(8)],
        pltpu.VMEM((8, CH), jnp.int32),
        [pltpu.VMEM((CH, HALF), jnp.float32) for _ in range(4)],
        pltpu.VMEM_SHARED((NPAD, HALF), jnp.float32),
        [pltpu.SemaphoreType.DMA for _ in range(8)],
        [pltpu.SemaphoreType.DMA for _ in range(8)],
        [pltpu.SemaphoreType.DMA for _ in range(4)],
        [pltpu.SemaphoreType.DMA for _ in range(4)],
    ],
)
def _scatter_kernel(hs_hbm, src2_hbm, dst_hbm, acc_hbm,
                    src_v, dst_v, rows_v, acc_sh, isems, dsems, gsems, ssems):
    c = lax.axis_index("c")
    s = lax.axis_index("s")
    rowbase = c * NPAD
    ebase = (c * NS + s) * EPTP
    dbase = s * EPTP
    pltpu.sync_copy(hs_hbm.at[pl.ds(rowbase + s * RPT, RPT)],
                    acc_sh.at[pl.ds(s * RPT, RPT)])
    plsc.subcore_barrier()

    def issue_idx(k, bi):
        pltpu.async_copy(src2_hbm.at[pl.ds(ebase + k * CH, CH)],
                         src_v[bi], isems[bi])
        pltpu.async_copy(dst_hbm.at[pl.ds(dbase + k * CH, CH)],
                         dst_v.at[bi], dsems[bi])

    def wait_idx_src(k, bi):
        pltpu.make_async_copy(src2_hbm.at[pl.ds(ebase + k * CH, CH)],
                              src_v[bi], isems[bi]).wait()

    def wait_idx_dst(k, bi):
        pltpu.make_async_copy(dst_hbm.at[pl.ds(dbase + k * CH, CH)],
                              dst_v.at[bi], dsems[bi]).wait()

    def issue_gather(bi, bg):
        pltpu.async_copy(hs_hbm.at[src_v[bi]], rows_v[bg], gsems[bg])

    def wait_gather(bi, bg):
        pltpu.make_async_copy(hs_hbm.at[src_v[bi]], rows_v[bg],
                              gsems[bg]).wait()

    def issue_scatter(bi, bg):
        pltpu.async_copy(rows_v[bg], acc_sh.at[dst_v.at[bi]], ssems[bg],
                         add=True)

    def wait_scatter(bi, bg):
        pltpu.make_async_copy(rows_v[bg], acc_sh.at[dst_v.at[bi]],
                              ssems[bg]).wait()

    for k in range(6):
        issue_idx(k, k)
    for k in range(2):
        wait_idx_src(k, k)
        issue_gather(k, k)

    def body(g, carry):
        for b in range(8):
            k = g * 8 + b
            b4 = b % 4

            @pl.when(k < NCH)
            def _():
                wait_gather(b, b4)
                wait_idx_dst(k, b)
                issue_scatter(b, b4)

                @pl.when(k >= 2)
                def _():
                    wait_scatter((b + 6) % 8, (b4 + 2) % 4)

                kn = k + 6

                @pl.when(kn < NCH)
                def _():
                    issue_idx(kn, (b + 6) % 8)

                kg = k + 2

                @pl.when(kg < NCH)
                def _():
                    wait_idx_src(kg, (b + 2) % 8)
                    issue_gather((b + 2) % 8, (b4 + 2) % 4)

        return carry

    lax.fori_loop(0, (NCH + 7) // 8, body, 0)
    for k in (NCH - 2, NCH - 1):
        wait_scatter(k % 8, k % 4)
    plsc.subcore_barrier()
    pltpu.sync_copy(acc_sh.at[pl.ds(s * RPT, RPT)],
                    acc_hbm.at[pl.ds(rowbase + s * RPT, RPT)])


def _finish_body(acc_ref, d0_ref, d1_ref, out_ref):
    dinv = lax.rsqrt(d0_ref[...] + d1_ref[...] + 1.0)
    out_ref[...] = jnp.maximum(acc_ref[...] * dinv, 0.0)


def _finish(acc, deg0, deg1):
    return pl.pallas_call(
        _finish_body,
        grid=(GN, NC),
        in_specs=[
            pl.BlockSpec((BN, HALF), lambda i, j: (j * GN + i, 0)),
            pl.BlockSpec((BN, 1), lambda i, j: (i, 0)),
            pl.BlockSpec((BN, 1), lambda i, j: (i, 0)),
        ],
        out_specs=pl.BlockSpec((BN, HALF), lambda i, j: (i, j)),
        out_shape=jax.ShapeDtypeStruct((N, D), jnp.float32),
    )(acc, deg0, deg1)


def kernel(x, edge_index, W, b):
    src = edge_index[0]
    dst = edge_index[1]
    spread = (N + jnp.arange(256, dtype=jnp.int32) % (NPAD - N))
    src2 = jnp.concatenate([src, src + NPAD])
    pad1 = jnp.broadcast_to(spread[: EWP - EW], (NC * NS, EWP - EW))
    dstdeg = jnp.concatenate(
        [dst.reshape(NC * NS, EW), pad1], axis=1
    ).reshape(NC * NS * NCHD, CHD)
    zeros_col = jnp.zeros((NPAD,), jnp.float32)
    ones_chunk = jnp.ones((CHD,), jnp.float32)
    b_row = b.reshape(1, D)

    deg2 = _deg_kernel(dstdeg, zeros_col, ones_chunk)
    deg0 = deg2[:NPAD].reshape(NPAD, 1)
    deg1 = deg2[NPAD:].reshape(NPAD, 1)
    hs = _matmul_scale(x, W, b_row, deg0, deg1)
    acc = _scatter_kernel(hs, src2, dst)
    return _finish(acc, deg0, deg1)


# speedup vs baseline: 19.0167x; 1.0028x over previous
import functools

import jax
import jax.numpy as jnp
from jax import lax
from jax.experimental import pallas as pl
from jax.experimental.pallas import tpu as pltpu
from jax.experimental.pallas import tpu_sc as plsc

N = 10000
D = 256
E = 160000
HALF = D // 2
NC = 2
NS = 16
NPAD = 10240
RPT = NPAD // NS

CH = 80
EPT = E // NS
EPTP = EPT
NCH = EPTP // CH
CHD = 128

EW = E // (NC * NS)
EWP = 5120
NCHD = EWP // CHD

BN = 512
GN = NPAD // BN

_mesh = plsc.VectorSubcoreMesh(core_axis_name="c", subcore_axis_name="s")


@functools.partial(
    pl.kernel,
    out_type=jax.ShapeDtypeStruct((NC * NPAD,), jnp.float32),
    mesh=_mesh,
    scratch_types=[
        pltpu.VMEM((NCHD, CHD), jnp.int32),
        pltpu.VMEM((CHD,), jnp.float32),
        pltpu.VMEM_SHARED((NPAD,), jnp.float32),
        [pltpu.SemaphoreType.DMA for _ in range(4)],
    ],
)
def _deg_kernel(dstdeg_hbm, zeros_hbm, ones_hbm, deg_hbm,
                idx_v, ones_v, deg_sh, sems):
    c = lax.axis_index("c")
    s = lax.axis_index("s")
    w = c * NS + s
    pltpu.sync_copy(zeros_hbm.at[pl.ds(s * RPT, RPT)], deg_sh.at[pl.ds(s * RPT, RPT)])
    pltpu.sync_copy(ones_hbm, ones_v)
    pltpu.sync_copy(dstdeg_hbm.at[pl.ds(w * NCHD, NCHD)], idx_v)
    plsc.subcore_barrier()

    for b in range(4):
        pltpu.async_copy(ones_v, deg_sh.at[idx_v.at[b]], sems[b], add=True)

    def body(g, carry):
        for b in range(4):
            k = g * 4 + b
            pltpu.make_async_copy(ones_v, deg_sh.at[idx_v.at[k]], sems[b]).wait()
            kn = k + 4

            @pl.when(kn < NCHD)
            def _():
                pltpu.async_copy(ones_v, deg_sh.at[idx_v.at[kn]], sems[b], add=True)

        return carry

    lax.fori_loop(0, NCHD // 4, body, 0)
    plsc.subcore_barrier()
    pltpu.sync_copy(deg_sh.at[pl.ds(s * RPT, RPT)],
                    deg_hbm.at[pl.ds(c * NPAD + s * RPT, RPT)])


def _matmul_body(x_ref, w_ref, b_ref, d0_ref, d1_ref, out_ref):
    h = jnp.dot(x_ref[...], w_ref[...], preferred_element_type=jnp.float32)
    h = h + b_ref[...]
    dinv = lax.rsqrt(d0_ref[...] + d1_ref[...] + 1.0)
    out_ref[...] = h * dinv


def _matmul_scale(x_p, w, b_row, deg0, deg1):
    return pl.pallas_call(
        _matmul_body,
        grid=(GN, NC),
        in_specs=[
            pl.BlockSpec((BN, D), lambda i, j: (i, 0)),
            pl.BlockSpec((D, HALF), lambda i, j: (0, j)),
            pl.BlockSpec((1, HALF), lambda i, j: (0, j)),
            pl.BlockSpec((BN, 1), lambda i, j: (i, 0)),
            pl.BlockSpec((BN, 1), lambda i, j: (i, 0)),
        ],
        out_specs=pl.BlockSpec((BN, HALF), lambda i, j: (j * GN + i, 0)),
        out_shape=jax.ShapeDtypeStruct((NC * NPAD, HALF), jnp.float32),
    )(x_p, w, b_row, deg0, deg1)


@functools.partial(
    pl.kernel,
    out_type=jax.ShapeDtypeStruct((NC * NPAD, HALF), jnp.float32),
    mesh=_mesh,
    scratch_types=[
        [pltpu.VMEM((CH,), jnp.int32) for _ in range(8)],
        pltpu.VMEM((8, CH), jnp.int32),
        [pltpu.VMEM((CH, HALF), jnp.float32) for _ in range(4)],
        pltpu.VMEM_SHARED((NPAD, HALF), jnp.float32),
        [pltpu.SemaphoreType.DMA for _ in range(8)],
        [pltpu.SemaphoreType.DMA for _ in range(8)],
        [pltpu.SemaphoreType.DMA for _ in range(4)],
        [pltpu.SemaphoreType.DMA for _ in range(4)],
    ],
)
def _scatter_kernel(hs_hbm, src2_hbm, dst_hbm, acc_hbm,
                    src_v, dst_v, rows_v, acc_sh, isems, dsems, gsems, ssems):
    c = lax.axis_index("c")
    s = lax.axis_index("s")
    rowbase = c * NPAD
    ebase = (c * NS + s) * EPTP
    dbase = s * EPTP

    def issue_idx(k, bi):
        pltpu.async_copy(src2_hbm.at[pl.ds(ebase + k * CH, CH)],
                         src_v[bi], isems[bi])
        pltpu.async_copy(dst_hbm.at[pl.ds(dbase + k * CH, CH)],
                         dst_v.at[bi], dsems[bi])

    def wait_idx_src(k, bi):
        pltpu.make_async_copy(src2_hbm.at[pl.ds(ebase + k * CH, CH)],
                              src_v[bi], isems[bi]).wait()

    def wait_idx_dst(k, bi):
        pltpu.make_async_copy(dst_hbm.at[pl.ds(dbase + k * CH, CH)],
                              dst_v.at[bi], dsems[bi]).wait()

    def issue_gather(bi, bg):
        pltpu.async_copy(hs_hbm.at[src_v[bi]], rows_v[bg], gsems[bg])

    def wait_gather(bi, bg):
        pltpu.make_async_copy(hs_hbm.at[src_v[bi]], rows_v[bg],
                              gsems[bg]).wait()

    def issue_scatter(bi, bg):
        pltpu.async_copy(rows_v[bg], acc_sh.at[dst_v.at[bi]], ssems[bg],
                         add=True)

    def wait_scatter(bi, bg):
        pltpu.make_async_copy(rows_v[bg], acc_sh.at[dst_v.at[bi]],
                              ssems[bg]).wait()

    for k in range(6):
        issue_idx(k, k)
    pltpu.sync_copy(hs_hbm.at[pl.ds(rowbase + s * RPT, RPT)],
                    acc_sh.at[pl.ds(s * RPT, RPT)])
    plsc.subcore_barrier()
    for k in range(2):
        wait_idx_src(k, k)
        issue_gather(k, k)

    def body(g, carry):
        for b in range(8):
            k = g * 8 + b
            b4 = b % 4

            @pl.when(k < NCH)
            def _():
                wait_gather(b, b4)
                wait_idx_dst(k, b)
                issue_scatter(b, b4)

                @pl.when(k >= 2)
                def _():
                    wait_scatter((b + 6) % 8, (b4 + 2) % 4)

                kn = k + 6

                @pl.when(kn < NCH)
                def _():
                    issue_idx(kn, (b + 6) % 8)

                kg = k + 2

                @pl.when(kg < NCH)
                def _():
                    wait_idx_src(kg, (b + 2) % 8)
                    issue_gather((b + 2) % 8, (b4 + 2) % 4)

        return carry

    lax.fori_loop(0, (NCH + 7) // 8, body, 0)
    for k in (NCH - 2, NCH - 1):
        wait_scatter(k % 8, k % 4)
    plsc.subcore_barrier()
    pltpu.sync_copy(acc_sh.at[pl.ds(s * RPT, RPT)],
                    acc_hbm.at[pl.ds(rowbase + s * RPT, RPT)])


def _finish_body(acc_ref, d0_ref, d1_ref, out_ref):
    dinv = lax.rsqrt(d0_ref[...] + d1_ref[...] + 1.0)
    out_ref[...] = jnp.maximum(acc_ref[...] * dinv, 0.0)


def _finish(acc, deg0, deg1):
    return pl.pallas_call(
        _finish_body,
        grid=(GN, NC),
        in_specs=[
            pl.BlockSpec((BN, HALF), lambda i, j: (j * GN + i, 0)),
            pl.BlockSpec((BN, 1), lambda i, j: (i, 0)),
            pl.BlockSpec((BN, 1), lambda i, j: (i, 0)),
        ],
        out_specs=pl.BlockSpec((BN, HALF), lambda i, j: (i, j)),
        out_shape=jax.ShapeDtypeStruct((N, D), jnp.float32),
    )(acc, deg0, deg1)


def kernel(x, edge_index, W, b):
    src = edge_index[0]
    dst = edge_index[1]
    spread = (N + jnp.arange(256, dtype=jnp.int32) % (NPAD - N))
    src2 = jnp.concatenate([src, src + NPAD])
    pad1 = jnp.broadcast_to(spread[: EWP - EW], (NC * NS, EWP - EW))
    dstdeg = jnp.concatenate(
        [dst.reshape(NC * NS, EW), pad1], axis=1
    ).reshape(NC * NS * NCHD, CHD)
    zeros_col = jnp.zeros((NPAD,), jnp.float32)
    ones_chunk = jnp.ones((CHD,), jnp.float32)
    b_row = b.reshape(1, D)

    deg2 = _deg_kernel(dstdeg, zeros_col, ones_chunk)
    deg0 = deg2[:NPAD].reshape(NPAD, 1)
    deg1 = deg2[NPAD:].reshape(NPAD, 1)
    hs = _matmul_scale(x, W, b_row, deg0, deg1)
    acc = _scatter_kernel(hs, src2, dst)
    return _finish(acc, deg0, deg1)
